# baseline (speedup 1.0000x reference)
import functools

import jax
import jax.numpy as jnp
from jax import lax
from jax.experimental import pallas as pl
from jax.experimental.pallas import tpu as pltpu
from jax.experimental.pallas import tpu_sc as plsc

NU = 50000
NT = 50000
EF = 400000
D_NUM, D_CAT, D_DES, D_TW = 5, 3, 768, 768
DIN = D_NUM + D_CAT + D_DES + D_TW
EMB = 128

NCORE = 2
NSUB = 16
BB = 128
GRP = 51
GB = 4
TPB = GRP * GB
EPAD = NSUB * TPB * BB
RPT = 3136
ACC_ROWS = NSUB * RPT
CHUNKS = 4
CW = 32
CNTW = 16

_f32 = jnp.float32


def _segsum_body(table, idxc, zrows, out_f, out_p,
                 acc, idx0, idx1, idx2, buf0, buf1, buf2, buf3,
                 isem0, isem1, isem2, gsem0, gsem1, gsem2, gsem3,
                 ssem0, ssem1, ssem2, ssem3):
  c = lax.axis_index("c")
  s = lax.axis_index("s")
  idxb = (idx0, idx1, idx2)
  isems = (isem0, isem1, isem2)
  bufs = (buf0, buf1, buf2, buf3)
  gsems = (gsem0, gsem1, gsem2, gsem3)
  ssems = (ssem0, ssem1, ssem2, ssem3)
  for ch in range(CHUNKS):
    pltpu.sync_copy(zrows, acc.at[pl.ds(s * RPT, RPT)])
    plsc.subcore_barrier()
    pltpu.async_copy(idxc.at[c, ch, s, 0], idx0, isem0)
    pltpu.make_async_copy(idxc.at[c, ch, s, 0], idx0, isem0).wait()
    pltpu.async_copy(idxc.at[c, ch, s, 1], idx1, isem1)
    pltpu.async_copy(table.at[idx0.at[0, 0]], buf0, gsem0)
    pltpu.async_copy(table.at[idx0.at[0, 1]], buf1, gsem1)

    def trip(i, carry):
      for hh in range(3):
        g = i * 3 + hh
        ib = idxb[hh]
        for b3 in range(GB):
          b = g * GB + b3
          k = b3
          k2 = (b3 + 2) % GB
          nh = (hh + (b3 + 2) // GB) % 3
          nrow = (b3 + 2) % GB

          if b3 == 0:
            @pl.when(g + 1 < GRP)
            def _():
              pltpu.make_async_copy(idxc.at[c, ch, s, 0],
                                    idxb[(hh + 1) % 3],
                                    isems[(hh + 1) % 3]).wait()

          @pl.when(jnp.logical_and(b >= 2, b + 2 < TPB))
          def _():
            pltpu.make_async_copy(
                bufs[k2], acc.at[ib.at[1, 0]], ssems[k2]).wait()

          if b3 == 1:
            @pl.when(g + 2 < GRP)
            def _():
              pltpu.async_copy(idxc.at[c, ch, s, g + 2],
                               idxb[(hh + 2) % 3], isems[(hh + 2) % 3])

          @pl.when(b + 2 < TPB)
          def _():
            pltpu.async_copy(table.at[idxb[nh].at[0, nrow]],
                             bufs[k2], gsems[k2])

          pltpu.make_async_copy(table.at[ib.at[0, 0]], bufs[k],
                                gsems[k]).wait()
          pltpu.async_copy(bufs[k], acc.at[ib.at[1, b3]], ssems[k], add=True)
      return carry

    lax.fori_loop(0, GRP // 3, trip, 0)
    for k in range(GB):
      pltpu.make_async_copy(bufs[k], acc.at[idx0.at[1, 0]], ssems[k]).wait()
    plsc.subcore_barrier()

    @pl.when(c == 0)
    def _():
      pltpu.sync_copy(acc.at[pl.ds(s * RPT, RPT)],
                      out_f.at[ch, pl.ds(s * RPT, RPT)])

    @pl.when(c == 1)
    def _():
      pltpu.sync_copy(acc.at[pl.ds(s * RPT, RPT)],
                      out_p.at[ch, pl.ds(s * RPT, RPT)])

    plsc.subcore_barrier()


def _count_body(dst, ones_h, zrows, out_f, out_p, acc, didx, onesb):
  c = lax.axis_index("c")
  s = lax.axis_index("s")
  pltpu.sync_copy(dst.at[c, s], didx)
  pltpu.sync_copy(ones_h, onesb)
  pltpu.sync_copy(zrows, acc.at[pl.ds(s * RPT, RPT)])
  plsc.subcore_barrier()

  def bat(b, carry):
    pltpu.sync_copy(onesb, acc.at[didx.at[b]], add=True)
    return carry

  lax.fori_loop(0, TPB, bat, 0)
  plsc.subcore_barrier()

  @pl.when(c == 0)
  def _():
    pltpu.sync_copy(acc.at[pl.ds(s * RPT, RPT)], out_f.at[pl.ds(s * RPT, RPT)])

  @pl.when(c == 1)
  def _():
    pltpu.sync_copy(acc.at[pl.ds(s * RPT, RPT)], out_p.at[pl.ds(s * RPT, RPT)])


@functools.lru_cache(maxsize=None)
def _build_segsum():
  return pl.kernel(
      _segsum_body,
      out_type=(jax.ShapeDtypeStruct((CHUNKS, ACC_ROWS, CW), _f32),
                jax.ShapeDtypeStruct((CHUNKS, ACC_ROWS, CW), _f32)),
      mesh=plsc.VectorSubcoreMesh(core_axis_name="c", subcore_axis_name="s"),
      compiler_params=pltpu.CompilerParams(use_tc_tiling_on_sc=False),
      scratch_types=(
          [pltpu.VMEM_SHARED((ACC_ROWS, CW), _f32)]
          + [pltpu.VMEM((2, GB, BB), jnp.int32) for _ in range(3)]
          + [pltpu.VMEM((BB, CW), _f32) for _ in range(4)]
          + [pltpu.SemaphoreType.DMA for _ in range(11)]
      ),
  )


@functools.lru_cache(maxsize=None)
def _build_counts():
  return pl.kernel(
      _count_body,
      out_type=(jax.ShapeDtypeStruct((ACC_ROWS, CNTW), _f32),
                jax.ShapeDtypeStruct((ACC_ROWS, CNTW), _f32)),
      mesh=plsc.VectorSubcoreMesh(core_axis_name="c", subcore_axis_name="s"),
      compiler_params=pltpu.CompilerParams(use_tc_tiling_on_sc=False),
      scratch_types=[
          pltpu.VMEM_SHARED((ACC_ROWS, CNTW), _f32),
          pltpu.VMEM((TPB, BB), jnp.int32),
          pltpu.VMEM((BB, CNTW), _f32),
      ],
  )


def _leaky(x):
  return jnp.where(x >= 0, x, 0.01 * x)


def _pre_user_body(x_ref, wb_ref, bb_ref, w2_ref, b2_ref, feat_ref, emb_ref):
  f = jnp.dot(x_ref[...], wb_ref[...], preferred_element_type=_f32)
  f = _leaky(f + bb_ref[...])
  e = jnp.dot(f, w2_ref[...], preferred_element_type=_f32)
  e = _leaky(e + b2_ref[...])
  feat_ref[...] = f
  emb_ref[...] = e


def _pre_tweet_body(x_ref, w_ref, b_ref, emb_ref):
  e = jnp.dot(x_ref[...], w_ref[...], preferred_element_type=_f32)
  emb_ref[...] = _leaky(e + b_ref[...])


def _layer_body(cnt_ref, sums_ref, emb_ref, wl_ref, wr_ref, bl_ref,
                out_ref, st_ref):
  i = pl.program_id(0)
  cnt = cnt_ref[...][:, 0:1]
  inv = 1.0 / jnp.maximum(cnt, 1.0)
  sfull = jnp.concatenate([sums_ref[k] for k in range(CHUNKS)], axis=1)
  o = jnp.dot(sfull * inv, wl_ref[...], preferred_element_type=_f32)
  o = o + jnp.dot(emb_ref[...], wr_ref[...], preferred_element_type=_f32)
  o = o + bl_ref[...]
  out_ref[...] = o

  @pl.when(i == 0)
  def _():
    st_ref[...] = jnp.zeros_like(st_ref)

  st_ref[0:1, :] += jnp.sum(o, axis=0, keepdims=True)
  st_ref[1:2, :] += jnp.sum(o * o, axis=0, keepdims=True)


def _bn_scale_shift(st_ref, g_ref, b_ref, n):
  s1 = st_ref[0:1, :]
  s2 = st_ref[1:2, :]
  m = s1 / n
  v = s2 / n - m * m
  scale = g_ref[...] / jnp.sqrt(v + 1e-5)
  shift = b_ref[...] - m * scale
  return scale, shift


def _bn_body(x_ref, st_ref, g_ref, b_ref, o_ref, *, n):
  scale, shift = _bn_scale_shift(st_ref, g_ref, b_ref, n)
  o_ref[...] = x_ref[...] * scale + shift


def _bn_concat_body(x_ref, st_ref, g_ref, b_ref, feat_ref, o_ref, *, n):
  scale, shift = _bn_scale_shift(st_ref, g_ref, b_ref, n)
  o_ref[...] = jnp.concatenate(
      [x_ref[...] * scale + shift, feat_ref[...]], axis=1)


_BU = 1000


def _pre_user(user_x, wb, bb, w2, b2):
  return pl.pallas_call(
      _pre_user_body,
      grid=(NU // _BU,),
      in_specs=[
          pl.BlockSpec((_BU, DIN), lambda i: (i, 0)),
          pl.BlockSpec((DIN, EMB), lambda i: (0, 0)),
          pl.BlockSpec((1, EMB), lambda i: (0, 0)),
          pl.BlockSpec((EMB, EMB), lambda i: (0, 0)),
          pl.BlockSpec((1, EMB), lambda i: (0, 0)),
      ],
      out_specs=[
          pl.BlockSpec((_BU, EMB), lambda i: (i, 0)),
          pl.BlockSpec((_BU, EMB), lambda i: (i, 0)),
      ],
      out_shape=[
          jax.ShapeDtypeStruct((NU, EMB), _f32),
          jax.ShapeDtypeStruct((NU, EMB), _f32),
      ],
  )(user_x, wb, bb, w2, b2)


def _pre_tweet(tweet_x, w, b):
  return pl.pallas_call(
      _pre_tweet_body,
      grid=(NT // _BU,),
      in_specs=[
          pl.BlockSpec((_BU, D_TW), lambda i: (i, 0)),
          pl.BlockSpec((D_TW, EMB), lambda i: (0, 0)),
          pl.BlockSpec((1, EMB), lambda i: (0, 0)),
      ],
      out_specs=pl.BlockSpec((_BU, EMB), lambda i: (i, 0)),
      out_shape=jax.ShapeDtypeStruct((NT, EMB), _f32),
  )(tweet_x, w, b)


def _layer(cnt, sums, emb, wl, wr, bl, n):
  return pl.pallas_call(
      _layer_body,
      grid=(n // _BU,),
      in_specs=[
          pl.BlockSpec((_BU, CNTW), lambda i: (i, 0)),
          pl.BlockSpec((CHUNKS, _BU, CW), lambda i: (0, i, 0)),
          pl.BlockSpec((_BU, EMB), lambda i: (i, 0)),
          pl.BlockSpec((EMB, EMB), lambda i: (0, 0)),
          pl.BlockSpec((EMB, EMB), lambda i: (0, 0)),
          pl.BlockSpec((1, EMB), lambda i: (0, 0)),
      ],
      out_specs=[
          pl.BlockSpec((_BU, EMB), lambda i: (i, 0)),
          pl.BlockSpec((8, EMB), lambda i: (0, 0)),
      ],
      out_shape=[
          jax.ShapeDtypeStruct((n, EMB), _f32),
          jax.ShapeDtypeStruct((8, EMB), _f32),
      ],
  )(cnt, sums, emb, wl, wr, bl)


def _bn_apply(x, st, g, b, n):
  return pl.pallas_call(
      functools.partial(_bn_body, n=float(n)),
      grid=(n // _BU,),
      in_specs=[
          pl.BlockSpec((_BU, EMB), lambda i: (i, 0)),
          pl.BlockSpec((8, EMB), lambda i: (0, 0)),
          pl.BlockSpec((1, EMB), lambda i: (0, 0)),
          pl.BlockSpec((1, EMB), lambda i: (0, 0)),
      ],
      out_specs=pl.BlockSpec((_BU, EMB), lambda i: (i, 0)),
      out_shape=jax.ShapeDtypeStruct((n, EMB), _f32),
  )(x, st, g, b)


def _bn_concat(x, st, g, b, feat, n):
  d = feat.shape[1]
  return pl.pallas_call(
      functools.partial(_bn_concat_body, n=float(n)),
      grid=(n // _BU,),
      in_specs=[
          pl.BlockSpec((_BU, EMB), lambda i: (i, 0)),
          pl.BlockSpec((8, EMB), lambda i: (0, 0)),
          pl.BlockSpec((1, EMB), lambda i: (0, 0)),
          pl.BlockSpec((1, EMB), lambda i: (0, 0)),
          pl.BlockSpec((_BU, d), lambda i: (i, 0)),
      ],
      out_specs=pl.BlockSpec((_BU, EMB + d), lambda i: (i, 0)),
      out_shape=jax.ShapeDtypeStruct((n, EMB + d), _f32),
  )(x, st, g, b, feat)


def _prep_edges(e, npad):
  src = jnp.concatenate([e[0], jnp.zeros((npad,), jnp.int32)])
  dst = jnp.concatenate([e[1], jnp.full((npad,), NU, jnp.int32)])
  src4c = (src * 4)[None, :] + jnp.arange(CHUNKS, dtype=jnp.int32)[:, None]
  src4c = src4c.reshape(CHUNKS, NSUB, GRP, 1, GB, BB)
  dstg = jnp.broadcast_to(dst.reshape(1, NSUB, GRP, 1, GB, BB),
                          src4c.shape)
  comb = jnp.concatenate([src4c, dstg], axis=3)
  return comb, dst.reshape(NSUB, TPB, BB)


def kernel(user_x, tweet_x, edge_follow, edge_post, params):
  p = params
  wb = jnp.zeros((DIN, EMB), _f32)
  wb = wb.at[0:D_NUM, 0:32].set(p['W_num'])
  wb = wb.at[D_NUM:D_NUM + D_CAT, 32:64].set(p['W_cat'])
  wb = wb.at[D_NUM + D_CAT:D_NUM + D_CAT + D_DES, 64:96].set(p['W_des'])
  wb = wb.at[D_NUM + D_CAT + D_DES:DIN, 96:128].set(p['W_tw'])
  bb = jnp.concatenate(
      [p['b_num'], p['b_cat'], p['b_des'], p['b_tw']])[None, :]

  feat, emb_u = _pre_user(user_x, wb, bb,
                          p['W_lin_user'], p['b_lin_user'][None, :])
  emb_t = _pre_tweet(tweet_x, p['W_lin_tweet'], p['b_lin_tweet'][None, :])

  sf4, df = _prep_edges(edge_follow, EPAD - edge_follow.shape[1])
  sp4, dp = _prep_edges(edge_post, EPAD - edge_post.shape[1])
  idxc = jnp.stack([sf4, sp4])
  dstst = jnp.stack([df, dp])
  zrows = jnp.zeros((RPT, CW), _f32)
  zcnt = jnp.zeros((RPT, CNTW), _f32)
  ones = jnp.ones((BB, CNTW), _f32)

  cnt_f, cnt_p = _build_counts()(dstst, ones, zcnt)

  out_user = out_tweet = None
  for l in range(2):
    table = emb_u.reshape(CHUNKS * NU, CW)
    sum_f, sum_p = _build_segsum()(table, idxc, zrows)
    out_u, st_u = _layer(cnt_f, sum_f, emb_u, p['Wl_follow_%d' % l],
                         p['Wr_follow_%d' % l], p['bl_follow_%d' % l][None, :],
                         NU)
    out_t, st_t = _layer(cnt_p, sum_p, emb_t, p['Wl_post_%d' % l],
                         p['Wr_post_%d' % l], p['bl_post_%d' % l][None, :],
                         NT)
    g = p['bn_g_%d' % l][None, :]
    b = p['bn_b_%d' % l][None, :]
    if l == 0:
      emb_u = _bn_apply(out_u, st_u, g, b, NU)
      emb_t = _bn_apply(out_t, st_t, g, b, NT)
    else:
      out_user = _bn_concat(out_u, st_u, g, b, feat, NU)
      out_tweet = _bn_concat(out_t, st_t, g, b, tweet_x, NT)
  return out_user, out_tweet


# speedup vs baseline: 1.8656x; 1.8656x over previous
import functools

import jax
import jax.numpy as jnp
from jax import lax
from jax.experimental import pallas as pl
from jax.experimental.pallas import tpu as pltpu
from jax.experimental.pallas import tpu_sc as plsc

NU = 50000
NT = 50000
EF = 400000
D_NUM, D_CAT, D_DES, D_TW = 5, 3, 768, 768
DIN = D_NUM + D_CAT + D_DES + D_TW
EMB = 128

NCORE = 2
NSUB = 16
BB = 128
GRP = 14
GB = 14
TPB = GRP * GB
EPAD = NSUB * TPB * BB
RPT = 3136
ACC_ROWS = NSUB * RPT
CHUNKS = 4
CW = 32
CNTW = 16

_f32 = jnp.float32


def _segsum_body(table, idxc, zrows, out_f, out_p,
                 acc, idx, buf0, buf1, sem0, sem1):
  c = lax.axis_index("c")
  s = lax.axis_index("s")
  bufs = (buf0, buf1)
  sems = (sem0, sem1)
  for ch in range(CHUNKS):
    pltpu.sync_copy(zrows, acc.at[pl.ds(s * RPT, RPT)])
    plsc.subcore_barrier()

    def grp(g, carry):
      pltpu.sync_copy(idxc.at[c, ch, s, g], idx)
      pltpu.async_copy(table.at[idx.at[0, 0]], buf0, sem0)
      pltpu.async_copy(table.at[idx.at[0, 1]], buf1, sem1)

      def bat(i, carry2):
        for k in range(2):
          b = i * 2 + k
          pltpu.make_async_copy(table.at[idx.at[0, 0]], bufs[k],
                                sems[k]).wait()
          pltpu.sync_copy(bufs[k], acc.at[idx.at[1, b]], add=True)

          @pl.when(b + 2 < GB)
          def _():
            pltpu.async_copy(table.at[idx.at[0, b + 2]], bufs[k], sems[k])
        return carry2

      lax.fori_loop(0, GB // 2, bat, 0)
      return carry

    lax.fori_loop(0, GRP, grp, 0)
    plsc.subcore_barrier()

    @pl.when(c == 0)
    def _():
      pltpu.sync_copy(acc.at[pl.ds(s * RPT, RPT)],
                      out_f.at[ch, pl.ds(s * RPT, RPT)])

    @pl.when(c == 1)
    def _():
      pltpu.sync_copy(acc.at[pl.ds(s * RPT, RPT)],
                      out_p.at[ch, pl.ds(s * RPT, RPT)])

    plsc.subcore_barrier()


def _count_body(dst, ones_h, zrows, out_f, out_p, acc, didx, onesb):
  c = lax.axis_index("c")
  s = lax.axis_index("s")
  pltpu.sync_copy(dst.at[c, s], didx)
  pltpu.sync_copy(ones_h, onesb)
  pltpu.sync_copy(zrows, acc.at[pl.ds(s * RPT, RPT)])
  plsc.subcore_barrier()

  def bat(b, carry):
    pltpu.sync_copy(onesb, acc.at[didx.at[b]], add=True)
    return carry

  lax.fori_loop(0, TPB, bat, 0)
  plsc.subcore_barrier()

  @pl.when(c == 0)
  def _():
    pltpu.sync_copy(acc.at[pl.ds(s * RPT, RPT)], out_f.at[pl.ds(s * RPT, RPT)])

  @pl.when(c == 1)
  def _():
    pltpu.sync_copy(acc.at[pl.ds(s * RPT, RPT)], out_p.at[pl.ds(s * RPT, RPT)])


@functools.lru_cache(maxsize=None)
def _build_segsum():
  return pl.kernel(
      _segsum_body,
      out_type=(jax.ShapeDtypeStruct((CHUNKS, ACC_ROWS, CW), _f32),
                jax.ShapeDtypeStruct((CHUNKS, ACC_ROWS, CW), _f32)),
      mesh=plsc.VectorSubcoreMesh(core_axis_name="c", subcore_axis_name="s"),
      compiler_params=pltpu.CompilerParams(use_tc_tiling_on_sc=False),
      scratch_types=(
          [pltpu.VMEM_SHARED((ACC_ROWS, CW), _f32)]
          + [pltpu.VMEM((2, GB, BB), jnp.int32)]
          + [pltpu.VMEM((BB, CW), _f32) for _ in range(2)]
          + [pltpu.SemaphoreType.DMA for _ in range(2)]
      ),
  )


@functools.lru_cache(maxsize=None)
def _build_counts():
  return pl.kernel(
      _count_body,
      out_type=(jax.ShapeDtypeStruct((ACC_ROWS, CNTW), _f32),
                jax.ShapeDtypeStruct((ACC_ROWS, CNTW), _f32)),
      mesh=plsc.VectorSubcoreMesh(core_axis_name="c", subcore_axis_name="s"),
      compiler_params=pltpu.CompilerParams(use_tc_tiling_on_sc=False),
      scratch_types=[
          pltpu.VMEM_SHARED((ACC_ROWS, CNTW), _f32),
          pltpu.VMEM((TPB, BB), jnp.int32),
          pltpu.VMEM((BB, CNTW), _f32),
      ],
  )


def _leaky(x):
  return jnp.where(x >= 0, x, 0.01 * x)


def _pre_user_body(x_ref, wb_ref, bb_ref, w2_ref, b2_ref, feat_ref, emb_ref):
  f = jnp.dot(x_ref[...], wb_ref[...], preferred_element_type=_f32)
  f = _leaky(f + bb_ref[...])
  e = jnp.dot(f, w2_ref[...], preferred_element_type=_f32)
  e = _leaky(e + b2_ref[...])
  feat_ref[...] = f
  emb_ref[...] = e


def _pre_tweet_body(x_ref, w_ref, b_ref, emb_ref):
  e = jnp.dot(x_ref[...], w_ref[...], preferred_element_type=_f32)
  emb_ref[...] = _leaky(e + b_ref[...])


def _layer_body(cnt_ref, sums_ref, emb_ref, wl_ref, wr_ref, bl_ref,
                out_ref, st_ref):
  i = pl.program_id(0)
  cnt = cnt_ref[...][:, 0:1]
  inv = 1.0 / jnp.maximum(cnt, 1.0)
  sfull = jnp.concatenate([sums_ref[k] for k in range(CHUNKS)], axis=1)
  o = jnp.dot(sfull * inv, wl_ref[...], preferred_element_type=_f32)
  o = o + jnp.dot(emb_ref[...], wr_ref[...], preferred_element_type=_f32)
  o = o + bl_ref[...]
  out_ref[...] = o

  @pl.when(i == 0)
  def _():
    st_ref[...] = jnp.zeros_like(st_ref)

  st_ref[0:1, :] += jnp.sum(o, axis=0, keepdims=True)
  st_ref[1:2, :] += jnp.sum(o * o, axis=0, keepdims=True)


def _bn_scale_shift(st_ref, g_ref, b_ref, n):
  s1 = st_ref[0:1, :]
  s2 = st_ref[1:2, :]
  m = s1 / n
  v = s2 / n - m * m
  scale = g_ref[...] / jnp.sqrt(v + 1e-5)
  shift = b_ref[...] - m * scale
  return scale, shift


def _bn_body(x_ref, st_ref, g_ref, b_ref, o_ref, *, n):
  scale, shift = _bn_scale_shift(st_ref, g_ref, b_ref, n)
  o_ref[...] = x_ref[...] * scale + shift


def _bn_concat_body(x_ref, st_ref, g_ref, b_ref, feat_ref, o_ref, *, n):
  scale, shift = _bn_scale_shift(st_ref, g_ref, b_ref, n)
  o_ref[...] = jnp.concatenate(
      [x_ref[...] * scale + shift, feat_ref[...]], axis=1)


_BU = 1000


def _pre_user(user_x, wb, bb, w2, b2):
  return pl.pallas_call(
      _pre_user_body,
      grid=(NU // _BU,),
      in_specs=[
          pl.BlockSpec((_BU, DIN), lambda i: (i, 0)),
          pl.BlockSpec((DIN, EMB), lambda i: (0, 0)),
          pl.BlockSpec((1, EMB), lambda i: (0, 0)),
          pl.BlockSpec((EMB, EMB), lambda i: (0, 0)),
          pl.BlockSpec((1, EMB), lambda i: (0, 0)),
      ],
      out_specs=[
          pl.BlockSpec((_BU, EMB), lambda i: (i, 0)),
          pl.BlockSpec((_BU, EMB), lambda i: (i, 0)),
      ],
      out_shape=[
          jax.ShapeDtypeStruct((NU, EMB), _f32),
          jax.ShapeDtypeStruct((NU, EMB), _f32),
      ],
  )(user_x, wb, bb, w2, b2)


def _pre_tweet(tweet_x, w, b):
  return pl.pallas_call(
      _pre_tweet_body,
      grid=(NT // _BU,),
      in_specs=[
          pl.BlockSpec((_BU, D_TW), lambda i: (i, 0)),
          pl.BlockSpec((D_TW, EMB), lambda i: (0, 0)),
          pl.BlockSpec((1, EMB), lambda i: (0, 0)),
      ],
      out_specs=pl.BlockSpec((_BU, EMB), lambda i: (i, 0)),
      out_shape=jax.ShapeDtypeStruct((NT, EMB), _f32),
  )(tweet_x, w, b)


def _layer(cnt, sums, emb, wl, wr, bl, n):
  return pl.pallas_call(
      _layer_body,
      grid=(n // _BU,),
      in_specs=[
          pl.BlockSpec((_BU, CNTW), lambda i: (i, 0)),
          pl.BlockSpec((CHUNKS, _BU, CW), lambda i: (0, i, 0)),
          pl.BlockSpec((_BU, EMB), lambda i: (i, 0)),
          pl.BlockSpec((EMB, EMB), lambda i: (0, 0)),
          pl.BlockSpec((EMB, EMB), lambda i: (0, 0)),
          pl.BlockSpec((1, EMB), lambda i: (0, 0)),
      ],
      out_specs=[
          pl.BlockSpec((_BU, EMB), lambda i: (i, 0)),
          pl.BlockSpec((8, EMB), lambda i: (0, 0)),
      ],
      out_shape=[
          jax.ShapeDtypeStruct((n, EMB), _f32),
          jax.ShapeDtypeStruct((8, EMB), _f32),
      ],
  )(cnt, sums, emb, wl, wr, bl)


def _bn_apply(x, st, g, b, n):
  return pl.pallas_call(
      functools.partial(_bn_body, n=float(n)),
      grid=(n // _BU,),
      in_specs=[
          pl.BlockSpec((_BU, EMB), lambda i: (i, 0)),
          pl.BlockSpec((8, EMB), lambda i: (0, 0)),
          pl.BlockSpec((1, EMB), lambda i: (0, 0)),
          pl.BlockSpec((1, EMB), lambda i: (0, 0)),
      ],
      out_specs=pl.BlockSpec((_BU, EMB), lambda i: (i, 0)),
      out_shape=jax.ShapeDtypeStruct((n, EMB), _f32),
  )(x, st, g, b)


def _bn_concat(x, st, g, b, feat, n):
  d = feat.shape[1]
  return pl.pallas_call(
      functools.partial(_bn_concat_body, n=float(n)),
      grid=(n // _BU,),
      in_specs=[
          pl.BlockSpec((_BU, EMB), lambda i: (i, 0)),
          pl.BlockSpec((8, EMB), lambda i: (0, 0)),
          pl.BlockSpec((1, EMB), lambda i: (0, 0)),
          pl.BlockSpec((1, EMB), lambda i: (0, 0)),
          pl.BlockSpec((_BU, d), lambda i: (i, 0)),
      ],
      out_specs=pl.BlockSpec((_BU, EMB + d), lambda i: (i, 0)),
      out_shape=jax.ShapeDtypeStruct((n, EMB + d), _f32),
  )(x, st, g, b, feat)


def _prep_edges(e, npad):
  src = jnp.concatenate([e[0], jnp.zeros((npad,), jnp.int32)])
  dst = jnp.concatenate([e[1], jnp.full((npad,), NU, jnp.int32)])
  src4c = (src * 4)[None, :] + jnp.arange(CHUNKS, dtype=jnp.int32)[:, None]
  src4c = src4c.reshape(CHUNKS, NSUB, GRP, 1, GB, BB)
  dstg = jnp.broadcast_to(dst.reshape(1, NSUB, GRP, 1, GB, BB),
                          src4c.shape)
  comb = jnp.concatenate([src4c, dstg], axis=3)
  return comb, dst.reshape(NSUB, TPB, BB)


def kernel(user_x, tweet_x, edge_follow, edge_post, params):
  p = params
  wb = jnp.zeros((DIN, EMB), _f32)
  wb = wb.at[0:D_NUM, 0:32].set(p['W_num'])
  wb = wb.at[D_NUM:D_NUM + D_CAT, 32:64].set(p['W_cat'])
  wb = wb.at[D_NUM + D_CAT:D_NUM + D_CAT + D_DES, 64:96].set(p['W_des'])
  wb = wb.at[D_NUM + D_CAT + D_DES:DIN, 96:128].set(p['W_tw'])
  bb = jnp.concatenate(
      [p['b_num'], p['b_cat'], p['b_des'], p['b_tw']])[None, :]

  feat, emb_u = _pre_user(user_x, wb, bb,
                          p['W_lin_user'], p['b_lin_user'][None, :])
  emb_t = _pre_tweet(tweet_x, p['W_lin_tweet'], p['b_lin_tweet'][None, :])

  sf4, df = _prep_edges(edge_follow, EPAD - edge_follow.shape[1])
  sp4, dp = _prep_edges(edge_post, EPAD - edge_post.shape[1])
  idxc = jnp.stack([sf4, sp4])
  dstst = jnp.stack([df, dp])
  zrows = jnp.zeros((RPT, CW), _f32)
  zcnt = jnp.zeros((RPT, CNTW), _f32)
  ones = jnp.ones((BB, CNTW), _f32)

  cnt_f, cnt_p = _build_counts()(dstst, ones, zcnt)

  out_user = out_tweet = None
  for l in range(2):
    table = emb_u.reshape(CHUNKS * NU, CW)
    sum_f, sum_p = _build_segsum()(table, idxc, zrows)
    out_u, st_u = _layer(cnt_f, sum_f, emb_u, p['Wl_follow_%d' % l],
                         p['Wr_follow_%d' % l], p['bl_follow_%d' % l][None, :],
                         NU)
    out_t, st_t = _layer(cnt_p, sum_p, emb_t, p['Wl_post_%d' % l],
                         p['Wr_post_%d' % l], p['bl_post_%d' % l][None, :],
                         NT)
    g = p['bn_g_%d' % l][None, :]
    b = p['bn_b_%d' % l][None, :]
    if l == 0:
      emb_u = _bn_apply(out_u, st_u, g, b, NU)
      emb_t = _bn_apply(out_t, st_t, g, b, NT)
    else:
      out_user = _bn_concat(out_u, st_u, g, b, feat, NU)
      out_tweet = _bn_concat(out_t, st_t, g, b, tweet_x, NT)
  return out_user, out_tweet


# speedup vs baseline: 1.8851x; 1.0104x over previous
import functools

import jax
import jax.numpy as jnp
from jax import lax
from jax.experimental import pallas as pl
from jax.experimental.pallas import tpu as pltpu
from jax.experimental.pallas import tpu_sc as plsc

NU = 50000
NT = 50000
EF = 400000
D_NUM, D_CAT, D_DES, D_TW = 5, 3, 768, 768
DIN = D_NUM + D_CAT + D_DES + D_TW
EMB = 128

NCORE = 2
NSUB = 16
BB = 128
GRP = 14
GB = 14
TPB = GRP * GB
EPAD = NSUB * TPB * BB
RPT = 3136
ACC_ROWS = NSUB * RPT
CHUNKS = 4
CW = 32
CNTW = 16

_f32 = jnp.float32


def _segsum_body(table, idxc, zrows, out_f, out_p,
                 acc, idx, buf0, buf1, sem0, sem1):
  c = lax.axis_index("c")
  s = lax.axis_index("s")
  bufs = (buf0, buf1)
  sems = (sem0, sem1)
  for ch in range(CHUNKS):
    pltpu.sync_copy(zrows, acc.at[pl.ds(s * RPT, RPT)])
    plsc.subcore_barrier()

    def grp(g, carry):
      pltpu.sync_copy(idxc.at[c, ch, s, g], idx)
      pltpu.async_copy(table.at[idx.at[0, 0]], buf0, sem0)
      pltpu.async_copy(table.at[idx.at[0, 1]], buf1, sem1)

      def bat(i, carry2):
        for k in range(2):
          b = i * 2 + k
          pltpu.make_async_copy(table.at[idx.at[0, 0]], bufs[k],
                                sems[k]).wait()
          pltpu.sync_copy(bufs[k], acc.at[idx.at[1, b]], add=True)

          @pl.when(b + 2 < GB)
          def _():
            pltpu.async_copy(table.at[idx.at[0, b + 2]], bufs[k], sems[k])
        return carry2

      lax.fori_loop(0, GB // 2, bat, 0)
      return carry

    lax.fori_loop(0, GRP, grp, 0)
    plsc.subcore_barrier()

    @pl.when(c == 0)
    def _():
      pltpu.sync_copy(acc.at[pl.ds(s * RPT, RPT)],
                      out_f.at[ch, pl.ds(s * RPT, RPT)])

    @pl.when(c == 1)
    def _():
      pltpu.sync_copy(acc.at[pl.ds(s * RPT, RPT)],
                      out_p.at[ch, pl.ds(s * RPT, RPT)])

    plsc.subcore_barrier()


def _count_body(dst, ones_h, zrows, out_f, out_p, acc, didx, onesb):
  c = lax.axis_index("c")
  s = lax.axis_index("s")
  pltpu.sync_copy(dst.at[c, s], didx)
  pltpu.sync_copy(ones_h, onesb)
  pltpu.sync_copy(zrows, acc.at[pl.ds(s * RPT, RPT)])
  plsc.subcore_barrier()

  def bat(b, carry):
    pltpu.sync_copy(onesb, acc.at[didx.at[b]], add=True)
    return carry

  lax.fori_loop(0, TPB, bat, 0)
  plsc.subcore_barrier()

  @pl.when(c == 0)
  def _():
    pltpu.sync_copy(acc.at[pl.ds(s * RPT, RPT)], out_f.at[pl.ds(s * RPT, RPT)])

  @pl.when(c == 1)
  def _():
    pltpu.sync_copy(acc.at[pl.ds(s * RPT, RPT)], out_p.at[pl.ds(s * RPT, RPT)])


@functools.lru_cache(maxsize=None)
def _build_segsum():
  return pl.kernel(
      _segsum_body,
      out_type=(jax.ShapeDtypeStruct((CHUNKS, ACC_ROWS, CW), _f32),
                jax.ShapeDtypeStruct((CHUNKS, ACC_ROWS, CW), _f32)),
      mesh=plsc.VectorSubcoreMesh(core_axis_name="c", subcore_axis_name="s"),
      compiler_params=pltpu.CompilerParams(use_tc_tiling_on_sc=False),
      scratch_types=(
          [pltpu.VMEM_SHARED((ACC_ROWS, CW), _f32)]
          + [pltpu.VMEM((2, GB, BB), jnp.int32)]
          + [pltpu.VMEM((BB, CW), _f32) for _ in range(2)]
          + [pltpu.SemaphoreType.DMA for _ in range(2)]
      ),
  )


@functools.lru_cache(maxsize=None)
def _build_counts():
  return pl.kernel(
      _count_body,
      out_type=(jax.ShapeDtypeStruct((ACC_ROWS, CNTW), _f32),
                jax.ShapeDtypeStruct((ACC_ROWS, CNTW), _f32)),
      mesh=plsc.VectorSubcoreMesh(core_axis_name="c", subcore_axis_name="s"),
      compiler_params=pltpu.CompilerParams(use_tc_tiling_on_sc=False),
      scratch_types=[
          pltpu.VMEM_SHARED((ACC_ROWS, CNTW), _f32),
          pltpu.VMEM((TPB, BB), jnp.int32),
          pltpu.VMEM((BB, CNTW), _f32),
      ],
  )


def _leaky(x):
  return jnp.where(x >= 0, x, 0.01 * x)


def _pre_user_body(x_ref, wb_ref, bb_ref, w2_ref, b2_ref, feat_ref, emb_ref):
  f = jnp.dot(x_ref[...], wb_ref[...], preferred_element_type=_f32)
  f = _leaky(f + bb_ref[...])
  e = jnp.dot(f, w2_ref[...], preferred_element_type=_f32)
  e = _leaky(e + b2_ref[...])
  feat_ref[...] = f
  emb_ref[...] = e


def _pre_tweet_body(x_ref, w_ref, b_ref, emb_ref):
  e = jnp.dot(x_ref[...], w_ref[...], preferred_element_type=_f32)
  emb_ref[...] = _leaky(e + b_ref[...])


def _layer_body(cnt_ref, sums_ref, emb_ref, wl_ref, wr_ref, bl_ref,
                out_ref, st_ref):
  i = pl.program_id(0)
  cnt = cnt_ref[...][:, 0:1]
  inv = 1.0 / jnp.maximum(cnt, 1.0)
  sfull = jnp.concatenate([sums_ref[k] for k in range(CHUNKS)], axis=1)
  o = jnp.dot(sfull * inv, wl_ref[...], preferred_element_type=_f32)
  o = o + jnp.dot(emb_ref[...], wr_ref[...], preferred_element_type=_f32)
  o = o + bl_ref[...]
  out_ref[...] = o

  @pl.when(i == 0)
  def _():
    st_ref[...] = jnp.zeros_like(st_ref)

  st_ref[0:1, :] += jnp.sum(o, axis=0, keepdims=True)
  st_ref[1:2, :] += jnp.sum(o * o, axis=0, keepdims=True)


def _bn_scale_shift(st_ref, g_ref, b_ref, n):
  s1 = st_ref[0:1, :]
  s2 = st_ref[1:2, :]
  m = s1 / n
  v = s2 / n - m * m
  scale = g_ref[...] / jnp.sqrt(v + 1e-5)
  shift = b_ref[...] - m * scale
  return scale, shift


def _bn_body(x_ref, st_ref, g_ref, b_ref, o_ref, *, n):
  scale, shift = _bn_scale_shift(st_ref, g_ref, b_ref, n)
  o_ref[...] = x_ref[...] * scale + shift


def _copy128_body(src_ref, o_ref):
  o_ref[...] = src_ref[...]


def _bn_into_body(x_ref, st_ref, g_ref, b_ref, buf_ref, o_ref, *, n):
  del buf_ref
  scale, shift = _bn_scale_shift(st_ref, g_ref, b_ref, n)
  o_ref[...] = x_ref[...] * scale + shift


_BU = 1000


def _pre_user(user_x, wb, bb, w2, b2):
  return pl.pallas_call(
      _pre_user_body,
      grid=(NU // _BU,),
      in_specs=[
          pl.BlockSpec((_BU, DIN), lambda i: (i, 0)),
          pl.BlockSpec((DIN, EMB), lambda i: (0, 0)),
          pl.BlockSpec((1, EMB), lambda i: (0, 0)),
          pl.BlockSpec((EMB, EMB), lambda i: (0, 0)),
          pl.BlockSpec((1, EMB), lambda i: (0, 0)),
      ],
      out_specs=[
          pl.BlockSpec((_BU, EMB), lambda i: (i, 0)),
          pl.BlockSpec((_BU, EMB), lambda i: (i, 0)),
      ],
      out_shape=[
          jax.ShapeDtypeStruct((NU, EMB), _f32),
          jax.ShapeDtypeStruct((NU, EMB), _f32),
      ],
  )(user_x, wb, bb, w2, b2)


def _pre_tweet(tweet_x, w, b):
  return pl.pallas_call(
      _pre_tweet_body,
      grid=(NT // _BU,),
      in_specs=[
          pl.BlockSpec((_BU, D_TW), lambda i: (i, 0)),
          pl.BlockSpec((D_TW, EMB), lambda i: (0, 0)),
          pl.BlockSpec((1, EMB), lambda i: (0, 0)),
      ],
      out_specs=pl.BlockSpec((_BU, EMB), lambda i: (i, 0)),
      out_shape=jax.ShapeDtypeStruct((NT, EMB), _f32),
  )(tweet_x, w, b)


def _layer(cnt, sums, emb, wl, wr, bl, n):
  return pl.pallas_call(
      _layer_body,
      grid=(n // _BU,),
      in_specs=[
          pl.BlockSpec((_BU, CNTW), lambda i: (i, 0)),
          pl.BlockSpec((CHUNKS, _BU, CW), lambda i: (0, i, 0)),
          pl.BlockSpec((_BU, EMB), lambda i: (i, 0)),
          pl.BlockSpec((EMB, EMB), lambda i: (0, 0)),
          pl.BlockSpec((EMB, EMB), lambda i: (0, 0)),
          pl.BlockSpec((1, EMB), lambda i: (0, 0)),
      ],
      out_specs=[
          pl.BlockSpec((_BU, EMB), lambda i: (i, 0)),
          pl.BlockSpec((8, EMB), lambda i: (0, 0)),
      ],
      out_shape=[
          jax.ShapeDtypeStruct((n, EMB), _f32),
          jax.ShapeDtypeStruct((8, EMB), _f32),
      ],
  )(cnt, sums, emb, wl, wr, bl)


def _bn_apply(x, st, g, b, n):
  return pl.pallas_call(
      functools.partial(_bn_body, n=float(n)),
      grid=(n // _BU,),
      in_specs=[
          pl.BlockSpec((_BU, EMB), lambda i: (i, 0)),
          pl.BlockSpec((8, EMB), lambda i: (0, 0)),
          pl.BlockSpec((1, EMB), lambda i: (0, 0)),
          pl.BlockSpec((1, EMB), lambda i: (0, 0)),
      ],
      out_specs=pl.BlockSpec((_BU, EMB), lambda i: (i, 0)),
      out_shape=jax.ShapeDtypeStruct((n, EMB), _f32),
  )(x, st, g, b)


def _prefill_tail(feat, n, dout):
  d = feat.shape[1]
  return pl.pallas_call(
      _copy128_body,
      grid=(n // _BU, d // EMB),
      in_specs=[pl.BlockSpec((_BU, EMB), lambda i, j: (i, j))],
      out_specs=pl.BlockSpec((_BU, EMB), lambda i, j: (i, j + 1)),
      out_shape=jax.ShapeDtypeStruct((n, dout), _f32),
  )(feat)


def _bn_into(x, st, g, b, buf, n):
  dout = buf.shape[1]
  return pl.pallas_call(
      functools.partial(_bn_into_body, n=float(n)),
      grid=(n // _BU,),
      in_specs=[
          pl.BlockSpec((_BU, EMB), lambda i: (i, 0)),
          pl.BlockSpec((8, EMB), lambda i: (0, 0)),
          pl.BlockSpec((1, EMB), lambda i: (0, 0)),
          pl.BlockSpec((1, EMB), lambda i: (0, 0)),
          pl.BlockSpec(memory_space=pl.ANY),
      ],
      out_specs=pl.BlockSpec((_BU, EMB), lambda i: (i, 0)),
      out_shape=jax.ShapeDtypeStruct((n, dout), _f32),
      input_output_aliases={4: 0},
  )(x, st, g, b, buf)


def _prep_edges(e, npad):
  src = jnp.concatenate([e[0], jnp.zeros((npad,), jnp.int32)])
  dst = jnp.concatenate([e[1], jnp.full((npad,), NU, jnp.int32)])
  src4c = (src * 4)[None, :] + jnp.arange(CHUNKS, dtype=jnp.int32)[:, None]
  src4c = src4c.reshape(CHUNKS, NSUB, GRP, 1, GB, BB)
  dstg = jnp.broadcast_to(dst.reshape(1, NSUB, GRP, 1, GB, BB),
                          src4c.shape)
  comb = jnp.concatenate([src4c, dstg], axis=3)
  return comb, dst.reshape(NSUB, TPB, BB)


def kernel(user_x, tweet_x, edge_follow, edge_post, params):
  p = params
  wb = jnp.zeros((DIN, EMB), _f32)
  wb = wb.at[0:D_NUM, 0:32].set(p['W_num'])
  wb = wb.at[D_NUM:D_NUM + D_CAT, 32:64].set(p['W_cat'])
  wb = wb.at[D_NUM + D_CAT:D_NUM + D_CAT + D_DES, 64:96].set(p['W_des'])
  wb = wb.at[D_NUM + D_CAT + D_DES:DIN, 96:128].set(p['W_tw'])
  bb = jnp.concatenate(
      [p['b_num'], p['b_cat'], p['b_des'], p['b_tw']])[None, :]

  feat, emb_u = _pre_user(user_x, wb, bb,
                          p['W_lin_user'], p['b_lin_user'][None, :])
  emb_t = _pre_tweet(tweet_x, p['W_lin_tweet'], p['b_lin_tweet'][None, :])
  ubuf = _prefill_tail(feat, NU, 2 * EMB)
  tbuf = _prefill_tail(tweet_x, NT, EMB + D_TW)

  sf4, df = _prep_edges(edge_follow, EPAD - edge_follow.shape[1])
  sp4, dp = _prep_edges(edge_post, EPAD - edge_post.shape[1])
  idxc = jnp.stack([sf4, sp4])
  dstst = jnp.stack([df, dp])
  zrows = jnp.zeros((RPT, CW), _f32)
  zcnt = jnp.zeros((RPT, CNTW), _f32)
  ones = jnp.ones((BB, CNTW), _f32)

  cnt_f, cnt_p = _build_counts()(dstst, ones, zcnt)

  out_user = out_tweet = None
  for l in range(2):
    table = emb_u.reshape(CHUNKS * NU, CW)
    sum_f, sum_p = _build_segsum()(table, idxc, zrows)
    out_u, st_u = _layer(cnt_f, sum_f, emb_u, p['Wl_follow_%d' % l],
                         p['Wr_follow_%d' % l], p['bl_follow_%d' % l][None, :],
                         NU)
    out_t, st_t = _layer(cnt_p, sum_p, emb_t, p['Wl_post_%d' % l],
                         p['Wr_post_%d' % l], p['bl_post_%d' % l][None, :],
                         NT)
    g = p['bn_g_%d' % l][None, :]
    b = p['bn_b_%d' % l][None, :]
    if l == 0:
      emb_u = _bn_apply(out_u, st_u, g, b, NU)
      emb_t = _bn_apply(out_t, st_t, g, b, NT)
    else:
      out_user = _bn_into(out_u, st_u, g, b, ubuf, NU)
      out_tweet = _bn_into(out_t, st_t, g, b, tbuf, NT)
  return out_user, out_tweet


# speedup vs baseline: 2.0204x; 1.0718x over previous
import functools

import jax
import jax.numpy as jnp
from jax import lax
from jax.experimental import pallas as pl
from jax.experimental.pallas import tpu as pltpu
from jax.experimental.pallas import tpu_sc as plsc

NU = 50000
NT = 50000
EF = 400000
D_NUM, D_CAT, D_DES, D_TW = 5, 3, 768, 768
DIN = D_NUM + D_CAT + D_DES + D_TW
EMB = 128

NCORE = 2
NSUB = 16
BB = 128
GRP = 7
GB = 28
TPB = GRP * GB
EPAD = NSUB * TPB * BB
RPT = 3136
ACC_ROWS = NSUB * RPT
CHUNKS = 4
CW = 32
CNTW = 16

_f32 = jnp.float32


ZR = 392


def _segsum_body(table, idxc, zrows, out_f, out_p,
                 acc, idx, buf0, buf1, zbuf, sem0, sem1):
  c = lax.axis_index("c")
  s = lax.axis_index("s")
  bufs = (buf0, buf1)
  sems = (sem0, sem1)
  pltpu.sync_copy(zrows, zbuf)
  for ch in range(CHUNKS):
    for z in range(RPT // ZR):
      pltpu.sync_copy(zbuf, acc.at[pl.ds(s * RPT + z * ZR, ZR)])
    plsc.subcore_barrier()

    def grp(g, carry):
      pltpu.sync_copy(idxc.at[c, ch, s, g], idx)
      pltpu.async_copy(table.at[idx.at[0, 0]], buf0, sem0)
      pltpu.async_copy(table.at[idx.at[0, 1]], buf1, sem1)

      def bat(i, carry2):
        for k in range(2):
          b = i * 2 + k
          pltpu.make_async_copy(table.at[idx.at[0, 0]], bufs[k],
                                sems[k]).wait()
          pltpu.sync_copy(bufs[k], acc.at[idx.at[1, b]], add=True)

          @pl.when(b + 2 < GB)
          def _():
            pltpu.async_copy(table.at[idx.at[0, b + 2]], bufs[k], sems[k])
        return carry2

      lax.fori_loop(0, GB // 2, bat, 0)
      return carry

    lax.fori_loop(0, GRP, grp, 0)
    plsc.subcore_barrier()

    @pl.when(c == 0)
    def _():
      pltpu.sync_copy(acc.at[pl.ds(s * RPT, RPT)],
                      out_f.at[ch, pl.ds(s * RPT, RPT)])

    @pl.when(c == 1)
    def _():
      pltpu.sync_copy(acc.at[pl.ds(s * RPT, RPT)],
                      out_p.at[ch, pl.ds(s * RPT, RPT)])

    plsc.subcore_barrier()


def _count_body(dst, ones_h, zrows, out_f, out_p, acc, didx, onesb):
  c = lax.axis_index("c")
  s = lax.axis_index("s")
  pltpu.sync_copy(dst.at[c, s], didx)
  pltpu.sync_copy(ones_h, onesb)
  pltpu.sync_copy(zrows, acc.at[pl.ds(s * RPT, RPT)])
  plsc.subcore_barrier()

  def bat(b, carry):
    pltpu.sync_copy(onesb, acc.at[didx.at[b]], add=True)
    return carry

  lax.fori_loop(0, TPB, bat, 0)
  plsc.subcore_barrier()

  @pl.when(c == 0)
  def _():
    pltpu.sync_copy(acc.at[pl.ds(s * RPT, RPT)], out_f.at[pl.ds(s * RPT, RPT)])

  @pl.when(c == 1)
  def _():
    pltpu.sync_copy(acc.at[pl.ds(s * RPT, RPT)], out_p.at[pl.ds(s * RPT, RPT)])


@functools.lru_cache(maxsize=None)
def _build_segsum():
  return pl.kernel(
      _segsum_body,
      out_type=(jax.ShapeDtypeStruct((CHUNKS, ACC_ROWS, CW), _f32),
                jax.ShapeDtypeStruct((CHUNKS, ACC_ROWS, CW), _f32)),
      mesh=plsc.VectorSubcoreMesh(core_axis_name="c", subcore_axis_name="s"),
      compiler_params=pltpu.CompilerParams(use_tc_tiling_on_sc=False),
      scratch_types=(
          [pltpu.VMEM_SHARED((ACC_ROWS, CW), _f32)]
          + [pltpu.VMEM((2, GB, BB), jnp.int32)]
          + [pltpu.VMEM((BB, CW), _f32) for _ in range(2)]
          + [pltpu.VMEM((ZR, CW), _f32)]
          + [pltpu.SemaphoreType.DMA for _ in range(2)]
      ),
  )


@functools.lru_cache(maxsize=None)
def _build_counts():
  return pl.kernel(
      _count_body,
      out_type=(jax.ShapeDtypeStruct((ACC_ROWS, CNTW), _f32),
                jax.ShapeDtypeStruct((ACC_ROWS, CNTW), _f32)),
      mesh=plsc.VectorSubcoreMesh(core_axis_name="c", subcore_axis_name="s"),
      compiler_params=pltpu.CompilerParams(use_tc_tiling_on_sc=False),
      scratch_types=[
          pltpu.VMEM_SHARED((ACC_ROWS, CNTW), _f32),
          pltpu.VMEM((TPB, BB), jnp.int32),
          pltpu.VMEM((BB, CNTW), _f32),
      ],
  )


def _leaky(x):
  return jnp.where(x >= 0, x, 0.01 * x)


def _pre_user_body(x_ref, wb_ref, bb_ref, w2_ref, b2_ref, feat_ref, emb_ref):
  f = jnp.dot(x_ref[...], wb_ref[...], preferred_element_type=_f32)
  f = _leaky(f + bb_ref[...])
  e = jnp.dot(f, w2_ref[...], preferred_element_type=_f32)
  e = _leaky(e + b2_ref[...])
  feat_ref[...] = f
  emb_ref[...] = e


def _pre_tweet_body(x_ref, w_ref, b_ref, emb_ref):
  e = jnp.dot(x_ref[...], w_ref[...], preferred_element_type=_f32)
  emb_ref[...] = _leaky(e + b_ref[...])


def _layer_body(cnt_ref, sums_ref, emb_ref, wl_ref, wr_ref, bl_ref,
                out_ref, st_ref):
  i = pl.program_id(0)
  cnt = cnt_ref[...][:, 0:1]
  inv = 1.0 / jnp.maximum(cnt, 1.0)
  sfull = jnp.concatenate([sums_ref[k] for k in range(CHUNKS)], axis=1)
  o = jnp.dot(sfull * inv, wl_ref[...], preferred_element_type=_f32)
  o = o + jnp.dot(emb_ref[...], wr_ref[...], preferred_element_type=_f32)
  o = o + bl_ref[...]
  out_ref[...] = o

  @pl.when(i == 0)
  def _():
    st_ref[...] = jnp.zeros_like(st_ref)

  st_ref[0:1, :] += jnp.sum(o, axis=0, keepdims=True)
  st_ref[1:2, :] += jnp.sum(o * o, axis=0, keepdims=True)


def _bn_scale_shift(st_ref, g_ref, b_ref, n):
  s1 = st_ref[0:1, :]
  s2 = st_ref[1:2, :]
  m = s1 / n
  v = s2 / n - m * m
  scale = g_ref[...] / jnp.sqrt(v + 1e-5)
  shift = b_ref[...] - m * scale
  return scale, shift


def _bn_body(x_ref, st_ref, g_ref, b_ref, o_ref, *, n):
  scale, shift = _bn_scale_shift(st_ref, g_ref, b_ref, n)
  o_ref[...] = x_ref[...] * scale + shift


def _copy128_body(src_ref, o_ref):
  o_ref[...] = src_ref[...]


def _bn_into_body(x_ref, st_ref, g_ref, b_ref, buf_ref, o_ref, *, n):
  del buf_ref
  scale, shift = _bn_scale_shift(st_ref, g_ref, b_ref, n)
  o_ref[...] = x_ref[...] * scale + shift


_BU = 2000


def _pre_user(user_x, wb, bb, w2, b2):
  return pl.pallas_call(
      _pre_user_body,
      grid=(NU // _BU,),
      in_specs=[
          pl.BlockSpec((_BU, DIN), lambda i: (i, 0)),
          pl.BlockSpec((DIN, EMB), lambda i: (0, 0)),
          pl.BlockSpec((1, EMB), lambda i: (0, 0)),
          pl.BlockSpec((EMB, EMB), lambda i: (0, 0)),
          pl.BlockSpec((1, EMB), lambda i: (0, 0)),
      ],
      out_specs=[
          pl.BlockSpec((_BU, EMB), lambda i: (i, 0)),
          pl.BlockSpec((_BU, EMB), lambda i: (i, 0)),
      ],
      out_shape=[
          jax.ShapeDtypeStruct((NU, EMB), _f32),
          jax.ShapeDtypeStruct((NU, EMB), _f32),
      ],
  )(user_x, wb, bb, w2, b2)


def _pre_tweet(tweet_x, w, b):
  return pl.pallas_call(
      _pre_tweet_body,
      grid=(NT // _BU,),
      in_specs=[
          pl.BlockSpec((_BU, D_TW), lambda i: (i, 0)),
          pl.BlockSpec((D_TW, EMB), lambda i: (0, 0)),
          pl.BlockSpec((1, EMB), lambda i: (0, 0)),
      ],
      out_specs=pl.BlockSpec((_BU, EMB), lambda i: (i, 0)),
      out_shape=jax.ShapeDtypeStruct((NT, EMB), _f32),
  )(tweet_x, w, b)


def _layer(cnt, sums, emb, wl, wr, bl, n):
  return pl.pallas_call(
      _layer_body,
      grid=(n // _BU,),
      in_specs=[
          pl.BlockSpec((_BU, CNTW), lambda i: (i, 0)),
          pl.BlockSpec((CHUNKS, _BU, CW), lambda i: (0, i, 0)),
          pl.BlockSpec((_BU, EMB), lambda i: (i, 0)),
          pl.BlockSpec((EMB, EMB), lambda i: (0, 0)),
          pl.BlockSpec((EMB, EMB), lambda i: (0, 0)),
          pl.BlockSpec((1, EMB), lambda i: (0, 0)),
      ],
      out_specs=[
          pl.BlockSpec((_BU, EMB), lambda i: (i, 0)),
          pl.BlockSpec((8, EMB), lambda i: (0, 0)),
      ],
      out_shape=[
          jax.ShapeDtypeStruct((n, EMB), _f32),
          jax.ShapeDtypeStruct((8, EMB), _f32),
      ],
  )(cnt, sums, emb, wl, wr, bl)


def _bn_apply(x, st, g, b, n):
  return pl.pallas_call(
      functools.partial(_bn_body, n=float(n)),
      grid=(n // _BU,),
      in_specs=[
          pl.BlockSpec((_BU, EMB), lambda i: (i, 0)),
          pl.BlockSpec((8, EMB), lambda i: (0, 0)),
          pl.BlockSpec((1, EMB), lambda i: (0, 0)),
          pl.BlockSpec((1, EMB), lambda i: (0, 0)),
      ],
      out_specs=pl.BlockSpec((_BU, EMB), lambda i: (i, 0)),
      out_shape=jax.ShapeDtypeStruct((n, EMB), _f32),
  )(x, st, g, b)


def _prefill_tail(feat, n, dout):
  d = feat.shape[1]
  return pl.pallas_call(
      _copy128_body,
      grid=(n // _BU, d // EMB),
      in_specs=[pl.BlockSpec((_BU, EMB), lambda i, j: (i, j))],
      out_specs=pl.BlockSpec((_BU, EMB), lambda i, j: (i, j + 1)),
      out_shape=jax.ShapeDtypeStruct((n, dout), _f32),
  )(feat)


def _bn_into(x, st, g, b, buf, n):
  dout = buf.shape[1]
  return pl.pallas_call(
      functools.partial(_bn_into_body, n=float(n)),
      grid=(n // _BU,),
      in_specs=[
          pl.BlockSpec((_BU, EMB), lambda i: (i, 0)),
          pl.BlockSpec((8, EMB), lambda i: (0, 0)),
          pl.BlockSpec((1, EMB), lambda i: (0, 0)),
          pl.BlockSpec((1, EMB), lambda i: (0, 0)),
          pl.BlockSpec(memory_space=pl.ANY),
      ],
      out_specs=pl.BlockSpec((_BU, EMB), lambda i: (i, 0)),
      out_shape=jax.ShapeDtypeStruct((n, dout), _f32),
      input_output_aliases={4: 0},
  )(x, st, g, b, buf)


def _prep_edges(e, npad):
  src = jnp.concatenate([e[0], jnp.zeros((npad,), jnp.int32)])
  dst = jnp.concatenate([e[1], jnp.full((npad,), NU, jnp.int32)])
  src4c = (src * 4)[None, :] + jnp.arange(CHUNKS, dtype=jnp.int32)[:, None]
  src4c = src4c.reshape(CHUNKS, NSUB, GRP, 1, GB, BB)
  dstg = jnp.broadcast_to(dst.reshape(1, NSUB, GRP, 1, GB, BB),
                          src4c.shape)
  comb = jnp.concatenate([src4c, dstg], axis=3)
  return comb, dst.reshape(NSUB, TPB, BB)


def kernel(user_x, tweet_x, edge_follow, edge_post, params):
  p = params
  wb = jnp.zeros((DIN, EMB), _f32)
  wb = wb.at[0:D_NUM, 0:32].set(p['W_num'])
  wb = wb.at[D_NUM:D_NUM + D_CAT, 32:64].set(p['W_cat'])
  wb = wb.at[D_NUM + D_CAT:D_NUM + D_CAT + D_DES, 64:96].set(p['W_des'])
  wb = wb.at[D_NUM + D_CAT + D_DES:DIN, 96:128].set(p['W_tw'])
  bb = jnp.concatenate(
      [p['b_num'], p['b_cat'], p['b_des'], p['b_tw']])[None, :]

  feat, emb_u = _pre_user(user_x, wb, bb,
                          p['W_lin_user'], p['b_lin_user'][None, :])
  emb_t = _pre_tweet(tweet_x, p['W_lin_tweet'], p['b_lin_tweet'][None, :])
  ubuf = _prefill_tail(feat, NU, 2 * EMB)
  tbuf = _prefill_tail(tweet_x, NT, EMB + D_TW)

  sf4, df = _prep_edges(edge_follow, EPAD - edge_follow.shape[1])
  sp4, dp = _prep_edges(edge_post, EPAD - edge_post.shape[1])
  idxc = jnp.stack([sf4, sp4])
  dstst = jnp.stack([df, dp])
  zrows = jnp.zeros((ZR, CW), _f32)
  zcnt = jnp.zeros((RPT, CNTW), _f32)
  ones = jnp.ones((BB, CNTW), _f32)

  cnt_f, cnt_p = _build_counts()(dstst, ones, zcnt)

  out_user = out_tweet = None
  for l in range(2):
    table = emb_u.reshape(CHUNKS * NU, CW)
    sum_f, sum_p = _build_segsum()(table, idxc, zrows)
    out_u, st_u = _layer(cnt_f, sum_f, emb_u, p['Wl_follow_%d' % l],
                         p['Wr_follow_%d' % l], p['bl_follow_%d' % l][None, :],
                         NU)
    out_t, st_t = _layer(cnt_p, sum_p, emb_t, p['Wl_post_%d' % l],
                         p['Wr_post_%d' % l], p['bl_post_%d' % l][None, :],
                         NT)
    g = p['bn_g_%d' % l][None, :]
    b = p['bn_b_%d' % l][None, :]
    if l == 0:
      emb_u = _bn_apply(out_u, st_u, g, b, NU)
      emb_t = _bn_apply(out_t, st_t, g, b, NT)
    else:
      out_user = _bn_into(out_u, st_u, g, b, ubuf, NU)
      out_tweet = _bn_into(out_t, st_t, g, b, tbuf, NT)
  return out_user, out_tweet
